# Initial kernel scaffold; baseline (speedup 1.0000x reference)
import jax
import jax.numpy as jnp
from jax.experimental import pallas as pl


def kernel(x, W_w, W_b, U_f, U_h):
    raise NotImplementedError("write your pallas kernel here")


# speedup vs baseline: 5.4345x; 5.4345x over previous
import functools

import numpy as np
import jax
import jax.numpy as jnp
from jax.experimental import pallas as pl

_H = 128
_ALIGN = 1024
_BIG_BLOCK = 512


@functools.lru_cache(maxsize=None)
def _tree_plan(n_nodes: int):
    max_level = int(np.floor(np.log2(n_nodes)))
    levels = []
    for lvl in range(max_level + 1):
        s = 2 ** lvl - 1
        e = min(2 ** (lvl + 1) - 1, n_nodes)
        levels.append((s, e - s))
    caps = []
    for lvl in range(max_level + 1):
        cap = levels[lvl][1]
        if lvl > 0:
            cap = max(cap, 2 * levels[lvl - 1][1])
        caps.append(cap)
    offs = []
    r = 0
    for c in caps:
        offs.append(r)
        r += ((c + _ALIGN - 1) // _ALIGN) * _ALIGN
    return max_level, levels, caps, offs, r


def _leaf_body(x_ref, hs_ref, wwt_ref, wb_ref, out_ref, *, bl, n):
    del hs_ref
    i = pl.program_id(0)
    wx = jnp.dot(x_ref[...], wwt_ref[...],
                 preferred_element_type=jnp.float32) + wb_ref[0:1, :]
    whx = wx[:, :_H]
    wfx = wx[:, _H:]
    f = jax.nn.sigmoid(wfx)
    hn = (1.0 - f) * jnp.tanh(whx)
    rows = i * bl + jax.lax.broadcasted_iota(jnp.int32, (bl, 1), 0)
    out_ref[...] = jnp.where(rows < n, hn, 0.0)


def _lvl_body(x_ref, slab_ref, wwt_ref, wb_ref, uf0_ref, uf1_ref,
              uh0_ref, uh1_ref, out_ref, *, bl, n):
    i = pl.program_id(0)
    wx = jnp.dot(x_ref[...], wwt_ref[...],
                 preferred_element_type=jnp.float32) + wb_ref[0:1, :]
    whx = wx[:, :_H]
    wfx = wx[:, _H:]
    slab = slab_ref[...]
    h3 = slab.reshape(bl, 2, _H)
    h0 = h3[:, 0, :]
    h1 = h3[:, 1, :]
    fpre = (jnp.dot(h0, uf0_ref[...], preferred_element_type=jnp.float32) +
            jnp.dot(h1, uf1_ref[...], preferred_element_type=jnp.float32))
    f = jax.nn.sigmoid(fpre + wfx)
    hc = jnp.tanh(whx +
                  jnp.dot(f * h0, uh0_ref[...],
                          preferred_element_type=jnp.float32) +
                  jnp.dot(f * h1, uh1_ref[...],
                          preferred_element_type=jnp.float32))
    hn = f * (h0 + h1) + (1.0 - f) * hc
    rows = i * bl + jax.lax.broadcasted_iota(jnp.int32, (bl, 1), 0)
    out_ref[...] = jnp.where(rows < n, hn, 0.0)


def _top_body(x_ref, slab_ref, wwt_ref, wb_ref, uf0_ref, uf1_ref,
              uh0_ref, uh1_ref, out_ref, *, levels, offs):
    xb = x_ref[...]
    wwt = wwt_ref[...]
    wb = wb_ref[0:1, :]
    uf0 = uf0_ref[...]
    uf1 = uf1_ref[...]
    uh0 = uh0_ref[...]
    uh1 = uh1_ref[...]
    hp = slab_ref[...]
    for l in range(8, -1, -1):
        s, n = levels[l]
        np8 = max(16, n)
        need = 2 * np8
        if hp.shape[0] < need:
            hp = jnp.concatenate(
                [hp, jnp.zeros((need - hp.shape[0], _H), jnp.float32)], axis=0)
        pairs = hp[:need].reshape(np8, 2, _H)
        h0 = pairs[:, 0, :]
        h1 = pairs[:, 1, :]
        xl = xb[s:s + np8, :]
        wx = jnp.dot(xl, wwt, preferred_element_type=jnp.float32) + wb
        whx = wx[:, :_H]
        wfx = wx[:, _H:]
        fpre = (jnp.dot(h0, uf0, preferred_element_type=jnp.float32) +
                jnp.dot(h1, uf1, preferred_element_type=jnp.float32))
        f = jax.nn.sigmoid(fpre + wfx)
        hc = jnp.tanh(whx +
                      jnp.dot(f * h0, uh0, preferred_element_type=jnp.float32) +
                      jnp.dot(f * h1, uh1, preferred_element_type=jnp.float32))
        hn = f * (h0 + h1) + (1.0 - f) * hc
        rows = jax.lax.broadcasted_iota(jnp.int32, (np8, 1), 0)
        hn = jnp.where(rows < n, hn, 0.0)
        out_ref[pl.ds(offs[l], np8), :] = hn
        hp = hn


def kernel(x, W_w, W_b, U_f, U_h):
    n_nodes = x.shape[0]
    max_level, levels, caps, offs, total_rows = _tree_plan(n_nodes)

    wwt = W_w.T
    wb8 = jnp.tile(W_b[None, :], (8, 1))
    uf0 = U_f[:, :_H].T
    uf1 = U_f[:, _H:].T
    uh0 = U_h[:, :_H].T
    uh1 = U_h[:, _H:].T

    h_store = jnp.zeros((total_rows, _H), dtype=jnp.float32)

    fuse_top = max_level >= 9
    bottom = 9 if fuse_top else 0

    for lvl in range(max_level, bottom - 1, -1):
        s, n = levels[lvl]
        bl = min(_BIG_BLOCK, 1 << max(3, (n - 1).bit_length()))
        nb = -(-n // bl)
        x_sl = jax.lax.slice(x, (s, 0), (s + n, x.shape[1]))
        if nb * bl != n:
            x_sl = jnp.pad(x_sl, ((0, nb * bl - n), (0, 0)))
        lvl_base = offs[lvl] // bl
        if lvl == max_level:
            h_store = pl.pallas_call(
                functools.partial(_leaf_body, bl=bl, n=n),
                grid=(nb,),
                in_specs=[
                    pl.BlockSpec((bl, _H), lambda i: (i, 0)),
                    pl.BlockSpec((bl, _H), lambda i: (0, 0)),
                    pl.BlockSpec((_H, 2 * _H), lambda i: (0, 0)),
                    pl.BlockSpec((8, 2 * _H), lambda i: (0, 0)),
                ],
                out_specs=pl.BlockSpec((bl, _H),
                                       lambda i, b=lvl_base: (b + i, 0)),
                out_shape=jax.ShapeDtypeStruct((total_rows, _H), jnp.float32),
                input_output_aliases={1: 0},
            )(x_sl, h_store, wwt, wb8)
        else:
            child_base = offs[lvl + 1] // (2 * bl)
            h_store = pl.pallas_call(
                functools.partial(_lvl_body, bl=bl, n=n),
                grid=(nb,),
                in_specs=[
                    pl.BlockSpec((bl, _H), lambda i: (i, 0)),
                    pl.BlockSpec((2 * bl, _H),
                                 lambda i, c=child_base: (c + i, 0)),
                    pl.BlockSpec((_H, 2 * _H), lambda i: (0, 0)),
                    pl.BlockSpec((8, 2 * _H), lambda i: (0, 0)),
                    pl.BlockSpec((_H, _H), lambda i: (0, 0)),
                    pl.BlockSpec((_H, _H), lambda i: (0, 0)),
                    pl.BlockSpec((_H, _H), lambda i: (0, 0)),
                    pl.BlockSpec((_H, _H), lambda i: (0, 0)),
                ],
                out_specs=pl.BlockSpec((bl, _H),
                                       lambda i, b=lvl_base: (b + i, 0)),
                out_shape=jax.ShapeDtypeStruct((total_rows, _H), jnp.float32),
                input_output_aliases={1: 0},
            )(x_sl, h_store, wwt, wb8, uf0, uf1, uh0, uh1)

    if fuse_top:
        h_store = pl.pallas_call(
            functools.partial(_top_body, levels=tuple(levels), offs=tuple(offs)),
            grid=(1,),
            in_specs=[
                pl.BlockSpec((512, _H), lambda i: (0, 0)),
                pl.BlockSpec((512, _H), lambda i, c=offs[9] // 512: (c, 0)),
                pl.BlockSpec((_H, 2 * _H), lambda i: (0, 0)),
                pl.BlockSpec((8, 2 * _H), lambda i: (0, 0)),
                pl.BlockSpec((_H, _H), lambda i: (0, 0)),
                pl.BlockSpec((_H, _H), lambda i: (0, 0)),
                pl.BlockSpec((_H, _H), lambda i: (0, 0)),
                pl.BlockSpec((_H, _H), lambda i: (0, 0)),
            ],
            out_specs=pl.BlockSpec((offs[9], _H), lambda i: (0, 0)),
            out_shape=jax.ShapeDtypeStruct((total_rows, _H), jnp.float32),
            input_output_aliases={1: 0},
        )(x, h_store, wwt, wb8, uf0, uf1, uh0, uh1)

    parts = [jax.lax.slice(h_store, (offs[l], 0), (offs[l] + levels[l][1], _H))
             for l in range(max_level + 1)]
    return jnp.concatenate(parts, axis=0)


# speedup vs baseline: 5.7147x; 1.0516x over previous
import functools

import numpy as np
import jax
import jax.numpy as jnp
from jax.experimental import pallas as pl
from jax.experimental.pallas import tpu as pltpu

_H = 128
_B = 512


def _plan(n_nodes):
    max_level = int(np.floor(np.log2(n_nodes)))
    levels = []
    for lvl in range(max_level + 1):
        s = 2 ** lvl - 1
        e = min(2 ** (lvl + 1) - 1, n_nodes)
        levels.append((s, e - s))
    return max_level, levels


def _mega_body(tbl_ref, xa_ref, xb_ref, wwt_ref, wb_ref, uf0_ref, uf1_ref,
               uh0_ref, uh1_ref, out_ref, scra_ref, scrb_ref, *, ja, jb):
    i = pl.program_id(0)
    child_row = tbl_ref[i, 2]
    cur_row = tbl_ref[i, 3]
    prev_is_a = tbl_ref[i, 4]
    row_limit = tbl_ref[i, 5]
    has_child = tbl_ref[i, 6]

    xa = xa_ref[...]
    xb = xb_ref[...]
    xl = jnp.concatenate([xa[_B - 1:, :], xb[:_B - 1, :]], axis=0)
    wx = jnp.dot(xl, wwt_ref[...], preferred_element_type=jnp.float32) \
        + wb_ref[0:1, :]
    whx = wx[:, :_H]
    wfx = wx[:, _H:]

    cra = jnp.where(prev_is_a == 1, child_row, 0)
    crb = jnp.where(prev_is_a == 1, 0, child_row)
    hca = scra_ref[pl.ds(cra, 2 * _B), :]
    hcb = scrb_ref[pl.ds(crb, 2 * _B), :]
    hc = jnp.where(prev_is_a == 1, hca, hcb)
    hc = jnp.where(has_child == 1, hc, 0.0)
    pairs = hc.reshape(_B, 2, _H)
    h0 = pairs[:, 0, :]
    h1 = pairs[:, 1, :]

    fpre = (jnp.dot(h0, uf0_ref[...], preferred_element_type=jnp.float32) +
            jnp.dot(h1, uf1_ref[...], preferred_element_type=jnp.float32))
    f = jax.nn.sigmoid(fpre + wfx)
    hcand = jnp.tanh(whx +
                     jnp.dot(f * h0, uh0_ref[...],
                             preferred_element_type=jnp.float32) +
                     jnp.dot(f * h1, uh1_ref[...],
                             preferred_element_type=jnp.float32))
    hn = f * (h0 + h1) + (1.0 - f) * hcand
    rows = jax.lax.broadcasted_iota(jnp.int32, (_B, 1), 0)
    hn = jnp.where(rows < row_limit, hn, 0.0)

    rowa = jnp.where(prev_is_a == 1, ja, cur_row)
    rowb = jnp.where(prev_is_a == 1, cur_row, jb)
    scra_ref[pl.ds(rowa, _B), :] = hn
    scrb_ref[pl.ds(rowb, _B), :] = hn
    out_ref[...] = hn


def _top9_body(x_ref, slab_ref, wwt_ref, wb_ref, uf0_ref, uf1_ref,
               uh0_ref, uh1_ref, out_ref, *, levels):
    xb = x_ref[...]
    wwt = wwt_ref[...]
    wb = wb_ref[0:1, :]
    uf0 = uf0_ref[...]
    uf1 = uf1_ref[...]
    uh0 = uh0_ref[...]
    uh1 = uh1_ref[...]
    hp = slab_ref[...]
    for l in range(8, -1, -1):
        s, n = levels[l]
        np8 = max(16, n)
        need = 2 * np8
        if hp.shape[0] < need:
            hp = jnp.concatenate(
                [hp, jnp.zeros((need - hp.shape[0], _H), jnp.float32)], axis=0)
        pairs = hp[:need].reshape(np8, 2, _H)
        h0 = pairs[:, 0, :]
        h1 = pairs[:, 1, :]
        xl = xb[s:s + np8, :]
        wx = jnp.dot(xl, wwt, preferred_element_type=jnp.float32) + wb
        whx = wx[:, :_H]
        wfx = wx[:, _H:]
        fpre = (jnp.dot(h0, uf0, preferred_element_type=jnp.float32) +
                jnp.dot(h1, uf1, preferred_element_type=jnp.float32))
        f = jax.nn.sigmoid(fpre + wfx)
        hcand = jnp.tanh(whx +
                         jnp.dot(f * h0, uh0,
                                 preferred_element_type=jnp.float32) +
                         jnp.dot(f * h1, uh1,
                                 preferred_element_type=jnp.float32))
        hn = f * (h0 + h1) + (1.0 - f) * hcand
        rows = jax.lax.broadcasted_iota(jnp.int32, (np8, 1), 0)
        hn = jnp.where(rows < n, hn, 0.0)
        out_ref[pl.ds(_B * l, np8), :] = hn
        hp = hn


def kernel(x, W_w, W_b, U_f, U_h):
    n_nodes = x.shape[0]
    max_level, levels = _plan(n_nodes)
    assert max_level >= 10

    wwt = W_w.T
    wb8 = jnp.tile(W_b[None, :], (8, 1))
    uf0 = U_f[:, :_H].T
    uf1 = U_f[:, _H:].T
    uh0 = U_h[:, :_H].T
    uh1 = U_h[:, _H:].T

    nbs = {}
    offs9 = {}
    r = 0
    for lvl in range(max_level, 8, -1):
        n = levels[lvl][1]
        nbs[lvl] = -(-n // _B)
        offs9[lvl] = r
        r += nbs[lvl] * _B
    total_rows = r

    rows_a = max(nbs[lvl] * _B for lvl in range(max_level, 8, -1)
                 if (max_level - lvl) % 2 == 0)
    rows_b = max(nbs[lvl] * _B for lvl in range(max_level, 8, -1)
                 if (max_level - lvl) % 2 == 1)
    ja, jb = rows_a, rows_b

    tbl = []
    for lvl in range(max_level, 8, -1):
        s, n = levels[lvl]
        cur_is_a = 1 if (max_level - lvl) % 2 == 0 else 0
        child_written = nbs[lvl + 1] * _B if lvl < max_level else 0
        for j in range(nbs[lvl]):
            xa_blk = (s + j * _B) // _B
            out_blk = offs9[lvl] // _B + j
            child_row = 2 * j * _B
            hasc = 1 if (lvl < max_level and child_row < child_written) else 0
            row_limit = min(_B, n - j * _B)
            tbl.append([xa_blk, out_blk, 0 if not hasc else child_row,
                        j * _B, 1 - cur_is_a, row_limit, hasc])
    tbl = np.asarray(tbl, dtype=np.int32)
    nsteps = tbl.shape[0]

    grid_spec = pltpu.PrefetchScalarGridSpec(
        num_scalar_prefetch=1,
        grid=(nsteps,),
        in_specs=[
            pl.BlockSpec((_B, _H), lambda i, t: (t[i, 0], 0)),
            pl.BlockSpec((_B, _H), lambda i, t: (t[i, 0] + 1, 0)),
            pl.BlockSpec((_H, 2 * _H), lambda i, t: (0, 0)),
            pl.BlockSpec((8, 2 * _H), lambda i, t: (0, 0)),
            pl.BlockSpec((_H, _H), lambda i, t: (0, 0)),
            pl.BlockSpec((_H, _H), lambda i, t: (0, 0)),
            pl.BlockSpec((_H, _H), lambda i, t: (0, 0)),
            pl.BlockSpec((_H, _H), lambda i, t: (0, 0)),
        ],
        out_specs=pl.BlockSpec((_B, _H), lambda i, t: (t[i, 1], 0)),
        scratch_shapes=[
            pltpu.VMEM((rows_a + _B, _H), jnp.float32),
            pltpu.VMEM((rows_b + _B, _H), jnp.float32),
        ],
    )
    nxb = -(-n_nodes // _B)
    x_pad = x
    if (nxb + 1) * _B > n_nodes:
        x_pad = jnp.pad(x, ((0, (nxb + 1) * _B - n_nodes), (0, 0)))

    h_main = pl.pallas_call(
        functools.partial(_mega_body, ja=ja, jb=jb),
        grid_spec=grid_spec,
        out_shape=jax.ShapeDtypeStruct((total_rows, _H), jnp.float32),
    )(tbl, x_pad, x_pad, wwt, wb8, uf0, uf1, uh0, uh1)

    lvl9_base = offs9[9] // _B
    h_top = pl.pallas_call(
        functools.partial(_top9_body, levels=tuple(levels)),
        grid=(1,),
        in_specs=[
            pl.BlockSpec((_B, _H), lambda i: (0, 0)),
            pl.BlockSpec((_B, _H), lambda i, c=lvl9_base: (c, 0)),
            pl.BlockSpec((_H, 2 * _H), lambda i: (0, 0)),
            pl.BlockSpec((8, 2 * _H), lambda i: (0, 0)),
            pl.BlockSpec((_H, _H), lambda i: (0, 0)),
            pl.BlockSpec((_H, _H), lambda i: (0, 0)),
            pl.BlockSpec((_H, _H), lambda i: (0, 0)),
            pl.BlockSpec((_H, _H), lambda i: (0, 0)),
        ],
        out_specs=pl.BlockSpec((_B * 9, _H), lambda i: (0, 0)),
        out_shape=jax.ShapeDtypeStruct((_B * 9, _H), jnp.float32),
    )(x, h_main, wwt, wb8, uf0, uf1, uh0, uh1)

    parts = [jax.lax.slice(h_top, (_B * l, 0), (_B * l + levels[l][1], _H))
             for l in range(9)]
    parts += [jax.lax.slice(h_main, (offs9[l], 0),
                            (offs9[l] + levels[l][1], _H))
              for l in range(9, max_level + 1)]
    return jnp.concatenate(parts, axis=0)


# speedup vs baseline: 6.1510x; 1.0763x over previous
import functools

import numpy as np
import jax
import jax.numpy as jnp
from jax.experimental import pallas as pl
from jax.experimental.pallas import tpu as pltpu

_H = 128
_B = 512


def _plan(n_nodes):
    max_level = int(np.floor(np.log2(n_nodes)))
    levels = []
    for lvl in range(max_level + 1):
        s = 2 ** lvl - 1
        e = min(2 ** (lvl + 1) - 1, n_nodes)
        levels.append((s, e - s))
    return max_level, levels


def _mega_body(tbl_ref, xa_ref, xb_ref, wwt_ref, wb_ref, uf0_ref, uf1_ref,
               uh0_ref, uh1_ref, out_ref, scra_ref, scrb_ref, *, ja, jb):
    i = pl.program_id(0)
    child_row = tbl_ref[i, 2]
    cur_row = tbl_ref[i, 3]
    prev_is_a = tbl_ref[i, 4]
    row_limit = tbl_ref[i, 5]
    has_child = tbl_ref[i, 6]

    xa = xa_ref[...]
    xb = xb_ref[...]
    xl = jnp.concatenate([xa[_B - 1:, :], xb[:_B - 1, :]], axis=0)
    wx = jnp.dot(xl, wwt_ref[...],
                 preferred_element_type=jnp.float32) + wb_ref[0:1, :]
    whx = wx[:, :_H]
    wfx = wx[:, _H:]

    cra = jnp.where(prev_is_a == 1, child_row, 0)
    crb = jnp.where(prev_is_a == 1, 0, child_row)
    hca = scra_ref[pl.ds(cra, 2 * _B), :]
    hcb = scrb_ref[pl.ds(crb, 2 * _B), :]
    hc = jnp.where(prev_is_a == 1, hca, hcb)
    hc = jnp.where(has_child == 1, hc, 0.0)
    pairs = hc.reshape(_B, 2, _H)
    h0 = pairs[:, 0, :]
    h1 = pairs[:, 1, :]

    fpre = (jnp.dot(h0, uf0_ref[...], preferred_element_type=jnp.float32) +
            jnp.dot(h1, uf1_ref[...], preferred_element_type=jnp.float32))
    f = jax.nn.sigmoid(fpre + wfx)
    hcand = jnp.tanh(whx +
                     jnp.dot(f * h0, uh0_ref[...],
                             preferred_element_type=jnp.float32) +
                     jnp.dot(f * h1, uh1_ref[...],
                             preferred_element_type=jnp.float32))
    hn = f * (h0 + h1) + (1.0 - f) * hcand
    rows = jax.lax.broadcasted_iota(jnp.int32, (_B, 1), 0)
    hn = jnp.where(rows < row_limit, hn, 0.0)

    rowa = jnp.where(prev_is_a == 1, ja, cur_row)
    rowb = jnp.where(prev_is_a == 1, cur_row, jb)
    scra_ref[pl.ds(rowa, _B), :] = hn
    scrb_ref[pl.ds(rowb, _B), :] = hn
    out_ref[...] = hn


def _top9_body(x_ref, slab_ref, wwt_ref, wb_ref, uf0_ref, uf1_ref,
               uh0_ref, uh1_ref, out_ref, *, levels):
    xb = x_ref[...]
    wwt = wwt_ref[...]
    wb = wb_ref[0:1, :]
    uf0 = uf0_ref[...]
    uf1 = uf1_ref[...]
    uh0 = uh0_ref[...]
    uh1 = uh1_ref[...]
    hp = slab_ref[...]
    for l in range(8, -1, -1):
        s, n = levels[l]
        np8 = max(16, n)
        need = 2 * np8
        if hp.shape[0] < need:
            hp = jnp.concatenate(
                [hp, jnp.zeros((need - hp.shape[0], _H), jnp.float32)], axis=0)
        pairs = hp[:need].reshape(np8, 2, _H)
        h0 = pairs[:, 0, :]
        h1 = pairs[:, 1, :]
        xl = xb[s:s + np8, :]
        wx = jnp.dot(xl, wwt, preferred_element_type=jnp.float32) + wb
        whx = wx[:, :_H]
        wfx = wx[:, _H:]
        fpre = (jnp.dot(h0, uf0, preferred_element_type=jnp.float32) +
                jnp.dot(h1, uf1, preferred_element_type=jnp.float32))
        f = jax.nn.sigmoid(fpre + wfx)
        hcand = jnp.tanh(whx +
                         jnp.dot(f * h0, uh0,
                                 preferred_element_type=jnp.float32) +
                         jnp.dot(f * h1, uh1,
                                 preferred_element_type=jnp.float32))
        hn = f * (h0 + h1) + (1.0 - f) * hcand
        rows = jax.lax.broadcasted_iota(jnp.int32, (np8, 1), 0)
        hn = jnp.where(rows < n, hn, 0.0)
        out_ref[pl.ds(_B * l, np8), :] = hn
        hp = hn


def kernel(x, W_w, W_b, U_f, U_h):
    n_nodes = x.shape[0]
    max_level, levels = _plan(n_nodes)
    assert max_level >= 10

    wwt = W_w.T
    wb8 = jnp.tile(W_b[None, :], (8, 1))
    uf0 = U_f[:, :_H].T
    uf1 = U_f[:, _H:].T
    uh0 = U_h[:, :_H].T
    uh1 = U_h[:, _H:].T

    nbs = {}
    offs9 = {}
    r = 0
    for lvl in range(max_level, 8, -1):
        n = levels[lvl][1]
        nbs[lvl] = -(-n // _B)
        offs9[lvl] = r
        r += nbs[lvl] * _B
    total_rows = r

    rows_a = max(nbs[lvl] * _B for lvl in range(max_level, 8, -1)
                 if (max_level - lvl) % 2 == 0)
    rows_b = max(nbs[lvl] * _B for lvl in range(max_level, 8, -1)
                 if (max_level - lvl) % 2 == 1)
    ja, jb = rows_a, rows_b

    tbl = []
    for lvl in range(max_level, 8, -1):
        s, n = levels[lvl]
        cur_is_a = 1 if (max_level - lvl) % 2 == 0 else 0
        child_written = nbs[lvl + 1] * _B if lvl < max_level else 0
        for j in range(nbs[lvl]):
            xa_blk = (s + j * _B) // _B
            out_blk = offs9[lvl] // _B + j
            child_row = 2 * j * _B
            hasc = 1 if (lvl < max_level and child_row < child_written) else 0
            row_limit = min(_B, n - j * _B)
            tbl.append([xa_blk, out_blk, 0 if not hasc else child_row,
                        j * _B, 1 - cur_is_a, row_limit, hasc])
    tbl = np.asarray(tbl, dtype=np.int32)
    nsteps = tbl.shape[0]

    grid_spec = pltpu.PrefetchScalarGridSpec(
        num_scalar_prefetch=1,
        grid=(nsteps,),
        in_specs=[
            pl.BlockSpec((_B, _H), lambda i, t: (t[i, 0], 0)),
            pl.BlockSpec((_B, _H),
                         lambda i, t, m=(n_nodes - 1) // _B:
                         (jnp.minimum(t[i, 0] + 1, m), 0)),
            pl.BlockSpec((_H, 2 * _H), lambda i, t: (0, 0)),
            pl.BlockSpec((8, 2 * _H), lambda i, t: (0, 0)),
            pl.BlockSpec((_H, _H), lambda i, t: (0, 0)),
            pl.BlockSpec((_H, _H), lambda i, t: (0, 0)),
            pl.BlockSpec((_H, _H), lambda i, t: (0, 0)),
            pl.BlockSpec((_H, _H), lambda i, t: (0, 0)),
        ],
        out_specs=pl.BlockSpec((_B, _H), lambda i, t: (t[i, 1], 0)),
        scratch_shapes=[
            pltpu.VMEM((rows_a + _B, _H), jnp.float32),
            pltpu.VMEM((rows_b + _B, _H), jnp.float32),
        ],
    )

    h_main = pl.pallas_call(
        functools.partial(_mega_body, ja=ja, jb=jb),
        grid_spec=grid_spec,
        out_shape=jax.ShapeDtypeStruct((total_rows, _H), jnp.float32),
    )(tbl, x, x, wwt, wb8, uf0, uf1, uh0, uh1)

    lvl9_base = offs9[9] // _B
    h_top = pl.pallas_call(
        functools.partial(_top9_body, levels=tuple(levels)),
        grid=(1,),
        in_specs=[
            pl.BlockSpec((_B, _H), lambda i: (0, 0)),
            pl.BlockSpec((_B, _H), lambda i, c=lvl9_base: (c, 0)),
            pl.BlockSpec((_H, 2 * _H), lambda i: (0, 0)),
            pl.BlockSpec((8, 2 * _H), lambda i: (0, 0)),
            pl.BlockSpec((_H, _H), lambda i: (0, 0)),
            pl.BlockSpec((_H, _H), lambda i: (0, 0)),
            pl.BlockSpec((_H, _H), lambda i: (0, 0)),
            pl.BlockSpec((_H, _H), lambda i: (0, 0)),
        ],
        out_specs=pl.BlockSpec((_B * 9, _H), lambda i: (0, 0)),
        out_shape=jax.ShapeDtypeStruct((_B * 9, _H), jnp.float32),
    )(x, h_main, wwt, wb8, uf0, uf1, uh0, uh1)

    parts = [jax.lax.slice(h_top, (_B * l, 0), (_B * l + levels[l][1], _H))
             for l in range(9)]
    parts += [jax.lax.slice(h_main, (offs9[l], 0),
                            (offs9[l] + levels[l][1], _H))
              for l in range(9, max_level + 1)]
    return jnp.concatenate(parts, axis=0)


# speedup vs baseline: 6.3385x; 1.0305x over previous
import functools

import numpy as np
import jax
import jax.numpy as jnp
from jax.experimental import pallas as pl
from jax.experimental.pallas import tpu as pltpu

_H = 128
_B = 512


def _plan(n_nodes):
    max_level = int(np.floor(np.log2(n_nodes)))
    levels = []
    for lvl in range(max_level + 1):
        s = 2 ** lvl - 1
        e = min(2 ** (lvl + 1) - 1, n_nodes)
        levels.append((s, e - s))
    return max_level, levels


def _mega_body(tbl_ref, xa_ref, xb_ref, wwt_ref, wb_ref, uf0_ref, uf1_ref,
               uh0_ref, uh1_ref, out_ref, scra_ref, scrb_ref, *, ja, jb):
    i = pl.program_id(0)
    child_row = tbl_ref[i, 2]
    cur_row = tbl_ref[i, 3]
    prev_is_a = tbl_ref[i, 4]
    row_limit = tbl_ref[i, 5]
    has_child = tbl_ref[i, 6]

    xa = xa_ref[...]
    xb = xb_ref[...]
    xl = jnp.concatenate([xa[7:, :], xb[:_B - 1, :]], axis=0)
    wx = jnp.dot(xl, wwt_ref[...],
                 preferred_element_type=jnp.float32) + wb_ref[0:1, :]
    whx = wx[:, :_H]
    wfx = wx[:, _H:]

    cra = jnp.where(prev_is_a == 1, child_row, 0)
    crb = jnp.where(prev_is_a == 1, 0, child_row)
    hca = scra_ref[pl.ds(cra, 2 * _B), :]
    hcb = scrb_ref[pl.ds(crb, 2 * _B), :]
    hc = jnp.where(prev_is_a == 1, hca, hcb)
    hc = jnp.where(has_child == 1, hc, 0.0)
    pairs = hc.reshape(_B, 2, _H)
    h0 = pairs[:, 0, :]
    h1 = pairs[:, 1, :]

    fpre = (jnp.dot(h0, uf0_ref[...], preferred_element_type=jnp.float32) +
            jnp.dot(h1, uf1_ref[...], preferred_element_type=jnp.float32))
    f = jax.nn.sigmoid(fpre + wfx)
    hcand = jnp.tanh(whx +
                     jnp.dot(f * h0, uh0_ref[...],
                             preferred_element_type=jnp.float32) +
                     jnp.dot(f * h1, uh1_ref[...],
                             preferred_element_type=jnp.float32))
    hn = f * (h0 + h1) + (1.0 - f) * hcand
    rows = jax.lax.broadcasted_iota(jnp.int32, (_B, 1), 0)
    hn = jnp.where(rows < row_limit, hn, 0.0)

    rowa = jnp.where(prev_is_a == 1, ja, cur_row)
    rowb = jnp.where(prev_is_a == 1, cur_row, jb)
    scra_ref[pl.ds(rowa, _B), :] = hn
    scrb_ref[pl.ds(rowb, _B), :] = hn
    out_ref[...] = hn


def _top9_body(x_ref, slab_ref, wwt_ref, wb_ref, uf0_ref, uf1_ref,
               uh0_ref, uh1_ref, out_ref, *, levels):
    xb = x_ref[...]
    wwt = wwt_ref[...]
    wb = wb_ref[0:1, :]
    uf0 = uf0_ref[...]
    uf1 = uf1_ref[...]
    uh0 = uh0_ref[...]
    uh1 = uh1_ref[...]
    hp = slab_ref[...]
    for l in range(8, -1, -1):
        s, n = levels[l]
        np8 = max(16, n)
        need = 2 * np8
        if hp.shape[0] < need:
            hp = jnp.concatenate(
                [hp, jnp.zeros((need - hp.shape[0], _H), jnp.float32)], axis=0)
        pairs = hp[:need].reshape(np8, 2, _H)
        h0 = pairs[:, 0, :]
        h1 = pairs[:, 1, :]
        xl = xb[s:s + np8, :]
        wx = jnp.dot(xl, wwt, preferred_element_type=jnp.float32) + wb
        whx = wx[:, :_H]
        wfx = wx[:, _H:]
        fpre = (jnp.dot(h0, uf0, preferred_element_type=jnp.float32) +
                jnp.dot(h1, uf1, preferred_element_type=jnp.float32))
        f = jax.nn.sigmoid(fpre + wfx)
        hcand = jnp.tanh(whx +
                         jnp.dot(f * h0, uh0,
                                 preferred_element_type=jnp.float32) +
                         jnp.dot(f * h1, uh1,
                                 preferred_element_type=jnp.float32))
        hn = f * (h0 + h1) + (1.0 - f) * hcand
        rows = jax.lax.broadcasted_iota(jnp.int32, (np8, 1), 0)
        hn = jnp.where(rows < n, hn, 0.0)
        out_ref[pl.ds(_B * l, np8), :] = hn
        hp = hn


def kernel(x, W_w, W_b, U_f, U_h):
    n_nodes = x.shape[0]
    max_level, levels = _plan(n_nodes)
    assert max_level >= 10

    wwt = W_w.T
    wb8 = jnp.tile(W_b[None, :], (8, 1))
    uf0 = U_f[:, :_H].T
    uf1 = U_f[:, _H:].T
    uh0 = U_h[:, :_H].T
    uh1 = U_h[:, _H:].T

    nbs = {}
    offs9 = {}
    r = 0
    for lvl in range(max_level, 8, -1):
        n = levels[lvl][1]
        nbs[lvl] = -(-n // _B)
        offs9[lvl] = r
        r += nbs[lvl] * _B
    total_rows = r

    rows_a = max(nbs[lvl] * _B for lvl in range(max_level, 8, -1)
                 if (max_level - lvl) % 2 == 0)
    rows_b = max(nbs[lvl] * _B for lvl in range(max_level, 8, -1)
                 if (max_level - lvl) % 2 == 1)
    ja, jb = rows_a, rows_b

    tbl = []
    for lvl in range(max_level, 8, -1):
        s, n = levels[lvl]
        cur_is_a = 1 if (max_level - lvl) % 2 == 0 else 0
        child_written = nbs[lvl + 1] * _B if lvl < max_level else 0
        for j in range(nbs[lvl]):
            xa_blk = (s + j * _B) // _B
            out_blk = offs9[lvl] // _B + j
            child_row = 2 * j * _B
            hasc = 1 if (lvl < max_level and child_row < child_written) else 0
            row_limit = min(_B, n - j * _B)
            tbl.append([xa_blk, out_blk, 0 if not hasc else child_row,
                        j * _B, 1 - cur_is_a, row_limit, hasc])
    tbl = np.asarray(tbl, dtype=np.int32)
    nsteps = tbl.shape[0]

    grid_spec = pltpu.PrefetchScalarGridSpec(
        num_scalar_prefetch=1,
        grid=(nsteps,),
        in_specs=[
            pl.BlockSpec((8, _H), lambda i, t: (64 * t[i, 0] + 63, 0)),
            pl.BlockSpec((_B, _H),
                         lambda i, t, m=(n_nodes - 1) // _B:
                         (jnp.minimum(t[i, 0] + 1, m), 0)),
            pl.BlockSpec((_H, 2 * _H), lambda i, t: (0, 0)),
            pl.BlockSpec((8, 2 * _H), lambda i, t: (0, 0)),
            pl.BlockSpec((_H, _H), lambda i, t: (0, 0)),
            pl.BlockSpec((_H, _H), lambda i, t: (0, 0)),
            pl.BlockSpec((_H, _H), lambda i, t: (0, 0)),
            pl.BlockSpec((_H, _H), lambda i, t: (0, 0)),
        ],
        out_specs=pl.BlockSpec((_B, _H), lambda i, t: (t[i, 1], 0)),
        scratch_shapes=[
            pltpu.VMEM((rows_a + _B, _H), jnp.float32),
            pltpu.VMEM((rows_b + _B, _H), jnp.float32),
        ],
    )

    h_main = pl.pallas_call(
        functools.partial(_mega_body, ja=ja, jb=jb),
        grid_spec=grid_spec,
        out_shape=jax.ShapeDtypeStruct((total_rows, _H), jnp.float32),
    )(tbl, x, x, wwt, wb8, uf0, uf1, uh0, uh1)

    lvl9_base = offs9[9] // _B
    h_top = pl.pallas_call(
        functools.partial(_top9_body, levels=tuple(levels)),
        grid=(1,),
        in_specs=[
            pl.BlockSpec((_B, _H), lambda i: (0, 0)),
            pl.BlockSpec((_B, _H), lambda i, c=lvl9_base: (c, 0)),
            pl.BlockSpec((_H, 2 * _H), lambda i: (0, 0)),
            pl.BlockSpec((8, 2 * _H), lambda i: (0, 0)),
            pl.BlockSpec((_H, _H), lambda i: (0, 0)),
            pl.BlockSpec((_H, _H), lambda i: (0, 0)),
            pl.BlockSpec((_H, _H), lambda i: (0, 0)),
            pl.BlockSpec((_H, _H), lambda i: (0, 0)),
        ],
        out_specs=pl.BlockSpec((_B * 9, _H), lambda i: (0, 0)),
        out_shape=jax.ShapeDtypeStruct((_B * 9, _H), jnp.float32),
    )(x, h_main, wwt, wb8, uf0, uf1, uh0, uh1)

    parts = [jax.lax.slice(h_top, (_B * l, 0), (_B * l + levels[l][1], _H))
             for l in range(9)]
    parts += [jax.lax.slice(h_main, (offs9[l], 0),
                            (offs9[l] + levels[l][1], _H))
              for l in range(9, max_level + 1)]
    return jnp.concatenate(parts, axis=0)


# speedup vs baseline: 6.6674x; 1.0519x over previous
import functools

import numpy as np
import jax
import jax.numpy as jnp
from jax.experimental import pallas as pl
from jax.experimental.pallas import tpu as pltpu

_H = 128
_B = 1024
_TB = 512


def _plan(n_nodes):
    max_level = int(np.floor(np.log2(n_nodes)))
    levels = []
    for lvl in range(max_level + 1):
        s = 2 ** lvl - 1
        e = min(2 ** (lvl + 1) - 1, n_nodes)
        levels.append((s, e - s))
    return max_level, levels


def _mega_body(tbl_ref, xa_ref, xb_ref, wwt_ref, wb_ref, uf0_ref, uf1_ref,
               uh0_ref, uh1_ref, out_ref, scra_ref, scrb_ref, *, ja, jb):
    i = pl.program_id(0)
    child_row = tbl_ref[i, 2]
    cur_row = tbl_ref[i, 3]
    prev_is_a = tbl_ref[i, 4]
    row_limit = tbl_ref[i, 5]
    has_child = tbl_ref[i, 6]

    xa = xa_ref[...]
    xb = xb_ref[...]
    xl = jnp.concatenate([xa[7:, :], xb[:_B - 1, :]], axis=0)
    wx = jnp.dot(xl, wwt_ref[...],
                 preferred_element_type=jnp.float32) + wb_ref[0:1, :]
    whx = wx[:, :_H]
    wfx = wx[:, _H:]

    cra = jnp.where(prev_is_a == 1, child_row, 0)
    crb = jnp.where(prev_is_a == 1, 0, child_row)
    hca = scra_ref[pl.ds(cra, 2 * _B), :]
    hcb = scrb_ref[pl.ds(crb, 2 * _B), :]
    hc = jnp.where(prev_is_a == 1, hca, hcb)
    hc = jnp.where(has_child == 1, hc, 0.0)
    pairs = hc.reshape(_B, 2, _H)
    h0 = pairs[:, 0, :]
    h1 = pairs[:, 1, :]

    fpre = (jnp.dot(h0, uf0_ref[...], preferred_element_type=jnp.float32) +
            jnp.dot(h1, uf1_ref[...], preferred_element_type=jnp.float32))
    f = jax.nn.sigmoid(fpre + wfx)
    hcand = jnp.tanh(whx +
                     jnp.dot(f * h0, uh0_ref[...],
                             preferred_element_type=jnp.float32) +
                     jnp.dot(f * h1, uh1_ref[...],
                             preferred_element_type=jnp.float32))
    hn = f * (h0 + h1) + (1.0 - f) * hcand
    rows = jax.lax.broadcasted_iota(jnp.int32, (_B, 1), 0)
    hn = jnp.where(rows < row_limit, hn, 0.0)

    rowa = jnp.where(prev_is_a == 1, ja, cur_row)
    rowb = jnp.where(prev_is_a == 1, cur_row, jb)
    scra_ref[pl.ds(rowa, _B), :] = hn
    scrb_ref[pl.ds(rowb, _B), :] = hn
    out_ref[...] = hn


def _top_body(x_ref, slab_ref, wwt_ref, wb_ref, uf0_ref, uf1_ref,
              uh0_ref, uh1_ref, out_ref, *, levels):
    xb = x_ref[...]
    wwt = wwt_ref[...]
    wb = wb_ref[0:1, :]
    uf0 = uf0_ref[...]
    uf1 = uf1_ref[...]
    uh0 = uh0_ref[...]
    uh1 = uh1_ref[...]
    hp = slab_ref[...]
    for l in range(9, -1, -1):
        s, n = levels[l]
        np8 = max(16, n)
        need = 2 * np8
        if hp.shape[0] < need:
            hp = jnp.concatenate(
                [hp, jnp.zeros((need - hp.shape[0], _H), jnp.float32)], axis=0)
        pairs = hp[:need].reshape(np8, 2, _H)
        h0 = pairs[:, 0, :]
        h1 = pairs[:, 1, :]
        xl = xb[s:s + np8, :]
        wx = jnp.dot(xl, wwt, preferred_element_type=jnp.float32) + wb
        whx = wx[:, :_H]
        wfx = wx[:, _H:]
        fpre = (jnp.dot(h0, uf0, preferred_element_type=jnp.float32) +
                jnp.dot(h1, uf1, preferred_element_type=jnp.float32))
        f = jax.nn.sigmoid(fpre + wfx)
        hcand = jnp.tanh(whx +
                         jnp.dot(f * h0, uh0,
                                 preferred_element_type=jnp.float32) +
                         jnp.dot(f * h1, uh1,
                                 preferred_element_type=jnp.float32))
        hn = f * (h0 + h1) + (1.0 - f) * hcand
        rows = jax.lax.broadcasted_iota(jnp.int32, (np8, 1), 0)
        hn = jnp.where(rows < n, hn, 0.0)
        out_ref[pl.ds(_TB * l, np8), :] = hn
        hp = hn


def kernel(x, W_w, W_b, U_f, U_h):
    n_nodes = x.shape[0]
    max_level, levels = _plan(n_nodes)
    assert max_level >= 10

    wwt = W_w.T
    wb8 = jnp.tile(W_b[None, :], (8, 1))
    uf0 = U_f[:, :_H].T
    uf1 = U_f[:, _H:].T
    uh0 = U_h[:, :_H].T
    uh1 = U_h[:, _H:].T

    nbs = {}
    offs = {}
    r = 0
    for lvl in range(max_level, 9, -1):
        n = levels[lvl][1]
        nbs[lvl] = -(-n // _B)
        offs[lvl] = r
        r += nbs[lvl] * _B
    total_rows = r

    rows_a = max(nbs[lvl] * _B for lvl in range(max_level, 9, -1)
                 if (max_level - lvl) % 2 == 0)
    rows_b = max(nbs[lvl] * _B for lvl in range(max_level, 9, -1)
                 if (max_level - lvl) % 2 == 1)
    ja, jb = rows_a, rows_b

    tbl = []
    for lvl in range(max_level, 9, -1):
        s, n = levels[lvl]
        cur_is_a = 1 if (max_level - lvl) % 2 == 0 else 0
        child_written = nbs[lvl + 1] * _B if lvl < max_level else 0
        for j in range(nbs[lvl]):
            xk = (s + 1) // _B + j
            out_blk = offs[lvl] // _B + j
            child_row = 2 * j * _B
            hasc = 1 if (lvl < max_level and child_row < child_written) else 0
            row_limit = min(_B, n - j * _B)
            tbl.append([xk, out_blk, 0 if not hasc else child_row,
                        j * _B, 1 - cur_is_a, row_limit, hasc])
    tbl = np.asarray(tbl, dtype=np.int32)
    nsteps = tbl.shape[0]

    grid_spec = pltpu.PrefetchScalarGridSpec(
        num_scalar_prefetch=1,
        grid=(nsteps,),
        in_specs=[
            pl.BlockSpec((8, _H), lambda i, t: (128 * t[i, 0] - 1, 0)),
            pl.BlockSpec((_B, _H),
                         lambda i, t, m=(n_nodes - 1) // _B:
                         (jnp.minimum(t[i, 0], m), 0)),
            pl.BlockSpec((_H, 2 * _H), lambda i, t: (0, 0)),
            pl.BlockSpec((8, 2 * _H), lambda i, t: (0, 0)),
            pl.BlockSpec((_H, _H), lambda i, t: (0, 0)),
            pl.BlockSpec((_H, _H), lambda i, t: (0, 0)),
            pl.BlockSpec((_H, _H), lambda i, t: (0, 0)),
            pl.BlockSpec((_H, _H), lambda i, t: (0, 0)),
        ],
        out_specs=pl.BlockSpec((_B, _H), lambda i, t: (t[i, 1], 0)),
        scratch_shapes=[
            pltpu.VMEM((rows_a + _B, _H), jnp.float32),
            pltpu.VMEM((rows_b + _B, _H), jnp.float32),
        ],
    )

    h_main = pl.pallas_call(
        functools.partial(_mega_body, ja=ja, jb=jb),
        grid_spec=grid_spec,
        out_shape=jax.ShapeDtypeStruct((total_rows, _H), jnp.float32),
    )(tbl, x, x, wwt, wb8, uf0, uf1, uh0, uh1)

    lvl10_base = offs[10] // _B
    h_top = pl.pallas_call(
        functools.partial(_top_body, levels=tuple(levels)),
        grid=(1,),
        in_specs=[
            pl.BlockSpec((_B, _H), lambda i: (0, 0)),
            pl.BlockSpec((_B, _H), lambda i, c=lvl10_base: (c, 0)),
            pl.BlockSpec((_H, 2 * _H), lambda i: (0, 0)),
            pl.BlockSpec((8, 2 * _H), lambda i: (0, 0)),
            pl.BlockSpec((_H, _H), lambda i: (0, 0)),
            pl.BlockSpec((_H, _H), lambda i: (0, 0)),
            pl.BlockSpec((_H, _H), lambda i: (0, 0)),
            pl.BlockSpec((_H, _H), lambda i: (0, 0)),
        ],
        out_specs=pl.BlockSpec((_TB * 10, _H), lambda i: (0, 0)),
        out_shape=jax.ShapeDtypeStruct((_TB * 10, _H), jnp.float32),
    )(x, h_main, wwt, wb8, uf0, uf1, uh0, uh1)

    parts = [jax.lax.slice(h_top, (_TB * l, 0), (_TB * l + levels[l][1], _H))
             for l in range(10)]
    parts += [jax.lax.slice(h_main, (offs[l], 0),
                            (offs[l] + levels[l][1], _H))
              for l in range(10, max_level + 1)]
    return jnp.concatenate(parts, axis=0)


# speedup vs baseline: 12.1865x; 1.8278x over previous
import functools

import numpy as np
import jax
import jax.numpy as jnp
from jax.experimental import pallas as pl
from jax.experimental.pallas import tpu as pltpu

_H = 128
_B = 1024


def _plan(n_nodes):
    max_level = int(np.floor(np.log2(n_nodes)))
    levels = []
    for lvl in range(max_level + 1):
        s = 2 ** lvl - 1
        e = min(2 ** (lvl + 1) - 1, n_nodes)
        levels.append((s, e - s))
    return max_level, levels


def _mega_body(tbl_ref, xa_ref, xb_ref, wwt_ref, wb_ref, uf0_ref, uf1_ref,
               uh0_ref, uh1_ref, out_ref, l10_ref, scra_ref, scrb_ref,
               c_ref, *, ja, jb, nsteps):
    i = pl.program_id(0)
    child_row = tbl_ref[i, 2]
    cur_row = tbl_ref[i, 3]
    prev_is_a = tbl_ref[i, 4]
    row_limit = tbl_ref[i, 5]
    has_child = tbl_ref[i, 6]

    xa = xa_ref[...]
    xb = xb_ref[...]
    xl = jnp.concatenate([xa[7:, :], xb[:_B - 1, :]], axis=0)
    wx = jnp.dot(xl, wwt_ref[...],
                 preferred_element_type=jnp.float32) + wb_ref[0:1, :]
    whx = wx[:, :_H]
    wfx = wx[:, _H:]

    cra = jnp.where(prev_is_a == 1, child_row, 0)
    crb = jnp.where(prev_is_a == 1, 0, child_row)
    hca = scra_ref[pl.ds(cra, 2 * _B), :]
    hcb = scrb_ref[pl.ds(crb, 2 * _B), :]
    hc = jnp.where(prev_is_a == 1, hca, hcb)
    hc = jnp.where(has_child == 1, hc, 0.0)
    pairs = hc.reshape(_B, 2, _H)
    h0 = pairs[:, 0, :]
    h1 = pairs[:, 1, :]

    fpre = (jnp.dot(h0, uf0_ref[...], preferred_element_type=jnp.float32) +
            jnp.dot(h1, uf1_ref[...], preferred_element_type=jnp.float32))
    f = jax.nn.sigmoid(fpre + wfx)
    hcand = jnp.tanh(whx +
                     jnp.dot(f * h0, uh0_ref[...],
                             preferred_element_type=jnp.float32) +
                     jnp.dot(f * h1, uh1_ref[...],
                             preferred_element_type=jnp.float32))
    hn = f * (h0 + h1) + (1.0 - f) * hcand
    rows = jax.lax.broadcasted_iota(jnp.int32, (_B, 1), 0)
    hn = jnp.where(rows < row_limit, hn, 0.0)

    rowa = jnp.where(prev_is_a == 1, ja, cur_row)
    rowb = jnp.where(prev_is_a == 1, cur_row, jb)
    scra_ref[pl.ds(rowa, _B), :] = hn
    scrb_ref[pl.ds(rowb, _B), :] = hn

    prev0 = c_ref[0:1, :]
    out_ref[...] = jnp.concatenate([hn[1:, :], prev0], axis=0)
    c_ref[0:1, :] = hn[0:1, :]

    @pl.when(i == nsteps - 1)
    def _():
        l10_ref[...] = hn


def _top_body(x_ref, hbuf_ref, slab_ref, wwt_ref, wb_ref, uf0_ref, uf1_ref,
              uh0_ref, uh1_ref, out_ref, *, levels):
    del hbuf_ref
    xb = x_ref[...]
    wwt = wwt_ref[...]
    wb = wb_ref[0:1, :]
    uf0 = uf0_ref[...]
    uf1 = uf1_ref[...]
    uh0 = uh0_ref[...]
    uh1 = uh1_ref[...]
    hp = slab_ref[...]
    out_ref[pl.ds(_B - 1, 1), :] = hp[0:1, :]
    for l in range(9, -1, -1):
        s, n = levels[l]
        np8 = max(16, n)
        need = 2 * np8
        if hp.shape[0] < need:
            hp = jnp.concatenate(
                [hp, jnp.zeros((need - hp.shape[0], _H), jnp.float32)], axis=0)
        pairs = hp[:need].reshape(np8, 2, _H)
        h0 = pairs[:, 0, :]
        h1 = pairs[:, 1, :]
        xl = xb[s:s + np8, :]
        wx = jnp.dot(xl, wwt, preferred_element_type=jnp.float32) + wb
        whx = wx[:, :_H]
        wfx = wx[:, _H:]
        fpre = (jnp.dot(h0, uf0, preferred_element_type=jnp.float32) +
                jnp.dot(h1, uf1, preferred_element_type=jnp.float32))
        f = jax.nn.sigmoid(fpre + wfx)
        hcand = jnp.tanh(whx +
                         jnp.dot(f * h0, uh0,
                                 preferred_element_type=jnp.float32) +
                         jnp.dot(f * h1, uh1,
                                 preferred_element_type=jnp.float32))
        hn = f * (h0 + h1) + (1.0 - f) * hcand
        rows = jax.lax.broadcasted_iota(jnp.int32, (np8, 1), 0)
        hn = jnp.where(rows < n, hn, 0.0)
        out_ref[pl.ds(s, n), :] = hn[:n, :]
        hp = hn


def kernel(x, W_w, W_b, U_f, U_h):
    n_nodes = x.shape[0]
    max_level, levels = _plan(n_nodes)
    assert max_level >= 10

    wwt = W_w.T
    wb8 = jnp.tile(W_b[None, :], (8, 1))
    uf0 = U_f[:, :_H].T
    uf1 = U_f[:, _H:].T
    uh0 = U_h[:, :_H].T
    uh1 = U_h[:, _H:].T

    nbs = {lvl: -(-levels[lvl][1] // _B) for lvl in range(max_level, 9, -1)}

    rows_a = max(nbs[lvl] * _B for lvl in range(max_level, 9, -1)
                 if (max_level - lvl) % 2 == 0)
    rows_b = max(nbs[lvl] * _B for lvl in range(max_level, 9, -1)
                 if (max_level - lvl) % 2 == 1)
    ja, jb = rows_a, rows_b

    tbl = []
    for lvl in range(max_level, 9, -1):
        s, n = levels[lvl]
        cur_is_a = 1 if (max_level - lvl) % 2 == 0 else 0
        child_written = nbs[lvl + 1] * _B if lvl < max_level else 0
        for j in range(nbs[lvl] - 1, -1, -1):
            xk = (s + 1) // _B + j
            child_row = 2 * j * _B
            hasc = 1 if (lvl < max_level and child_row < child_written) else 0
            row_limit = min(_B, n - j * _B)
            tbl.append([xk, 0, 0 if not hasc else child_row,
                        j * _B, 1 - cur_is_a, row_limit, hasc])
    tbl = np.asarray(tbl, dtype=np.int32)
    nsteps = tbl.shape[0]

    grid_spec = pltpu.PrefetchScalarGridSpec(
        num_scalar_prefetch=1,
        grid=(nsteps,),
        in_specs=[
            pl.BlockSpec((8, _H), lambda i, t: (128 * t[i, 0] - 1, 0)),
            pl.BlockSpec((_B, _H),
                         lambda i, t, m=(n_nodes - 1) // _B:
                         (jnp.minimum(t[i, 0], m), 0)),
            pl.BlockSpec((_H, 2 * _H), lambda i, t: (0, 0)),
            pl.BlockSpec((8, 2 * _H), lambda i, t: (0, 0)),
            pl.BlockSpec((_H, _H), lambda i, t: (0, 0)),
            pl.BlockSpec((_H, _H), lambda i, t: (0, 0)),
            pl.BlockSpec((_H, _H), lambda i, t: (0, 0)),
            pl.BlockSpec((_H, _H), lambda i, t: (0, 0)),
        ],
        out_specs=[
            pl.BlockSpec((_B, _H), lambda i, t: (t[i, 0], 0)),
            pl.BlockSpec((_B, _H), lambda i, t: (0, 0)),
        ],
        scratch_shapes=[
            pltpu.VMEM((rows_a + _B, _H), jnp.float32),
            pltpu.VMEM((rows_b + _B, _H), jnp.float32),
            pltpu.VMEM((8, _H), jnp.float32),
        ],
    )

    h_buf, lvl10 = pl.pallas_call(
        functools.partial(_mega_body, ja=ja, jb=jb, nsteps=nsteps),
        grid_spec=grid_spec,
        out_shape=[jax.ShapeDtypeStruct((n_nodes, _H), jnp.float32),
                   jax.ShapeDtypeStruct((_B, _H), jnp.float32)],
    )(tbl, x, x, wwt, wb8, uf0, uf1, uh0, uh1)

    out = pl.pallas_call(
        functools.partial(_top_body, levels=tuple(levels)),
        grid=(1,),
        in_specs=[
            pl.BlockSpec((_B, _H), lambda i: (0, 0)),
            pl.BlockSpec((_B, _H), lambda i: (0, 0)),
            pl.BlockSpec((_B, _H), lambda i: (0, 0)),
            pl.BlockSpec((_H, 2 * _H), lambda i: (0, 0)),
            pl.BlockSpec((8, 2 * _H), lambda i: (0, 0)),
            pl.BlockSpec((_H, _H), lambda i: (0, 0)),
            pl.BlockSpec((_H, _H), lambda i: (0, 0)),
            pl.BlockSpec((_H, _H), lambda i: (0, 0)),
            pl.BlockSpec((_H, _H), lambda i: (0, 0)),
        ],
        out_specs=pl.BlockSpec((_B, _H), lambda i: (0, 0)),
        out_shape=jax.ShapeDtypeStruct((n_nodes, _H), jnp.float32),
        input_output_aliases={1: 0},
    )(x, h_buf, lvl10, wwt, wb8, uf0, uf1, uh0, uh1)
    return out


# speedup vs baseline: 18.2346x; 1.4963x over previous
import functools

import numpy as np
import jax
import jax.numpy as jnp
from jax.experimental import pallas as pl
from jax.experimental.pallas import tpu as pltpu

_H = 128
_B = 1024


def _plan(n_nodes):
    max_level = int(np.floor(np.log2(n_nodes)))
    levels = []
    for lvl in range(max_level + 1):
        s = 2 ** lvl - 1
        e = min(2 ** (lvl + 1) - 1, n_nodes)
        levels.append((s, e - s))
    return max_level, levels


def _mega_body(tbl_ref, xa_ref, xb_ref, wwt_ref, wb_ref, uf0_ref, uf1_ref,
               uh0_ref, uh1_ref, out_ref, l10_ref, scre_ref, scro_ref,
               c_ref, *, zbase, nsteps):
    i = pl.program_id(0)
    wbase = tbl_ref[i, 1]
    rbase = tbl_ref[i, 2]
    row_limit = tbl_ref[i, 3]

    @pl.when(i == 0)
    def _():
        scre_ref[pl.ds(zbase, _B), :] = jnp.zeros((_B, _H), jnp.float32)
        scro_ref[pl.ds(zbase, _B), :] = jnp.zeros((_B, _H), jnp.float32)

    xa = xa_ref[...]
    xb = xb_ref[...]
    xl = jnp.concatenate([xa[7:, :], xb[:_B - 1, :]], axis=0)
    wx = jnp.dot(xl, wwt_ref[...],
                 preferred_element_type=jnp.float32) + wb_ref[0:1, :]
    whx = wx[:, :_H]
    wfx = wx[:, _H:]

    h0 = scre_ref[pl.ds(rbase, _B), :]
    h1 = scro_ref[pl.ds(rbase, _B), :]

    fpre = (jnp.dot(h0, uf0_ref[...], preferred_element_type=jnp.float32) +
            jnp.dot(h1, uf1_ref[...], preferred_element_type=jnp.float32))
    f = jax.nn.sigmoid(fpre + wfx)
    hcand = jnp.tanh(whx +
                     jnp.dot(f * h0, uh0_ref[...],
                             preferred_element_type=jnp.float32) +
                     jnp.dot(f * h1, uh1_ref[...],
                             preferred_element_type=jnp.float32))
    hn = f * (h0 + h1) + (1.0 - f) * hcand
    rows = jax.lax.broadcasted_iota(jnp.int32, (_B, 1), 0)
    hn = jnp.where(rows < row_limit, hn, 0.0)

    hsplit = hn.reshape(_B // 2, 2, _H)
    scre_ref[pl.ds(wbase, _B // 2), :] = hsplit[:, 0, :]
    scro_ref[pl.ds(wbase, _B // 2), :] = hsplit[:, 1, :]

    prev0 = c_ref[0:1, :]
    out_ref[...] = jnp.concatenate([hn[1:, :], prev0], axis=0)
    c_ref[0:1, :] = hn[0:1, :]

    @pl.when(i == nsteps - 1)
    def _():
        l10_ref[...] = hn


def _top_body(x_ref, hbuf_ref, slab_ref, wwt_ref, wb_ref, uf0_ref, uf1_ref,
              uh0_ref, uh1_ref, out_ref, *, levels):
    del hbuf_ref
    xb = x_ref[...]
    wwt = wwt_ref[...]
    wb = wb_ref[0:1, :]
    uf0 = uf0_ref[...]
    uf1 = uf1_ref[...]
    uh0 = uh0_ref[...]
    uh1 = uh1_ref[...]
    hp = slab_ref[...]
    out_ref[pl.ds(_B - 1, 1), :] = hp[0:1, :]
    for l in range(9, -1, -1):
        s, n = levels[l]
        np8 = max(16, n)
        need = 2 * np8
        if hp.shape[0] < need:
            hp = jnp.concatenate(
                [hp, jnp.zeros((need - hp.shape[0], _H), jnp.float32)], axis=0)
        pairs = hp[:need].reshape(np8, 2, _H)
        h0 = pairs[:, 0, :]
        h1 = pairs[:, 1, :]
        xl = xb[s:s + np8, :]
        wx = jnp.dot(xl, wwt, preferred_element_type=jnp.float32) + wb
        whx = wx[:, :_H]
        wfx = wx[:, _H:]
        fpre = (jnp.dot(h0, uf0, preferred_element_type=jnp.float32) +
                jnp.dot(h1, uf1, preferred_element_type=jnp.float32))
        f = jax.nn.sigmoid(fpre + wfx)
        hcand = jnp.tanh(whx +
                         jnp.dot(f * h0, uh0,
                                 preferred_element_type=jnp.float32) +
                         jnp.dot(f * h1, uh1,
                                 preferred_element_type=jnp.float32))
        hn = f * (h0 + h1) + (1.0 - f) * hcand
        rows = jax.lax.broadcasted_iota(jnp.int32, (np8, 1), 0)
        hn = jnp.where(rows < n, hn, 0.0)
        out_ref[pl.ds(s, n), :] = hn[:n, :]
        hp = hn


def kernel(x, W_w, W_b, U_f, U_h):
    n_nodes = x.shape[0]
    max_level, levels = _plan(n_nodes)
    assert max_level >= 10

    wwt = W_w.T
    wb8 = jnp.tile(W_b[None, :], (8, 1))
    uf0 = U_f[:, :_H].T
    uf1 = U_f[:, _H:].T
    uh0 = U_h[:, :_H].T
    uh1 = U_h[:, _H:].T

    nbs = {lvl: -(-levels[lvl][1] // _B) for lvl in range(max_level, 9, -1)}

    capE = max(nbs[lvl] * _B // 2 for lvl in range(max_level, 9, -1))
    zbase = 2 * capE

    tbl = []
    for lvl in range(max_level, 9, -1):
        s, n = levels[lvl]
        pcur = (max_level - lvl) % 2
        child_half = nbs[lvl + 1] * _B // 2 if lvl < max_level else 0
        for j in range(nbs[lvl] - 1, -1, -1):
            xk = (s + 1) // _B + j
            hasc = lvl < max_level and (j + 1) * _B <= child_half
            row_limit = min(_B, n - j * _B)
            tbl.append([xk, pcur * capE + j * (_B // 2),
                        ((1 - pcur) * capE + j * _B) if hasc else zbase,
                        row_limit])
    tbl = np.asarray(tbl, dtype=np.int32)
    nsteps = tbl.shape[0]

    grid_spec = pltpu.PrefetchScalarGridSpec(
        num_scalar_prefetch=1,
        grid=(nsteps,),
        in_specs=[
            pl.BlockSpec((8, _H), lambda i, t: (128 * t[i, 0] - 1, 0)),
            pl.BlockSpec((_B, _H),
                         lambda i, t, m=(n_nodes - 1) // _B:
                         (jnp.minimum(t[i, 0], m), 0)),
            pl.BlockSpec((_H, 2 * _H), lambda i, t: (0, 0)),
            pl.BlockSpec((8, 2 * _H), lambda i, t: (0, 0)),
            pl.BlockSpec((_H, _H), lambda i, t: (0, 0)),
            pl.BlockSpec((_H, _H), lambda i, t: (0, 0)),
            pl.BlockSpec((_H, _H), lambda i, t: (0, 0)),
            pl.BlockSpec((_H, _H), lambda i, t: (0, 0)),
        ],
        out_specs=[
            pl.BlockSpec((_B, _H), lambda i, t: (t[i, 0], 0)),
            pl.BlockSpec((_B, _H), lambda i, t: (0, 0)),
        ],
        scratch_shapes=[
            pltpu.VMEM((2 * capE + _B, _H), jnp.float32),
            pltpu.VMEM((2 * capE + _B, _H), jnp.float32),
            pltpu.VMEM((8, _H), jnp.float32),
        ],
    )

    h_buf, lvl10 = pl.pallas_call(
        functools.partial(_mega_body, zbase=zbase, nsteps=nsteps),
        grid_spec=grid_spec,
        out_shape=[jax.ShapeDtypeStruct((n_nodes, _H), jnp.float32),
                   jax.ShapeDtypeStruct((_B, _H), jnp.float32)],
    )(tbl, x, x, wwt, wb8, uf0, uf1, uh0, uh1)

    out = pl.pallas_call(
        functools.partial(_top_body, levels=tuple(levels)),
        grid=(1,),
        in_specs=[
            pl.BlockSpec((_B, _H), lambda i: (0, 0)),
            pl.BlockSpec((_B, _H), lambda i: (0, 0)),
            pl.BlockSpec((_B, _H), lambda i: (0, 0)),
            pl.BlockSpec((_H, 2 * _H), lambda i: (0, 0)),
            pl.BlockSpec((8, 2 * _H), lambda i: (0, 0)),
            pl.BlockSpec((_H, _H), lambda i: (0, 0)),
            pl.BlockSpec((_H, _H), lambda i: (0, 0)),
            pl.BlockSpec((_H, _H), lambda i: (0, 0)),
            pl.BlockSpec((_H, _H), lambda i: (0, 0)),
        ],
        out_specs=pl.BlockSpec((_B, _H), lambda i: (0, 0)),
        out_shape=jax.ShapeDtypeStruct((n_nodes, _H), jnp.float32),
        input_output_aliases={1: 0},
    )(x, h_buf, lvl10, wwt, wb8, uf0, uf1, uh0, uh1)
    return out


# speedup vs baseline: 18.2843x; 1.0027x over previous
import functools

import numpy as np
import jax
import jax.numpy as jnp
from jax.experimental import pallas as pl
from jax.experimental.pallas import tpu as pltpu

_H = 128
_B = 1024


def _plan(n_nodes):
    max_level = int(np.floor(np.log2(n_nodes)))
    levels = []
    for lvl in range(max_level + 1):
        s = 2 ** lvl - 1
        e = min(2 ** (lvl + 1) - 1, n_nodes)
        levels.append((s, e - s))
    return max_level, levels


def _mega_body(tbl_ref, xa_ref, xb_ref, wwt_ref, wb_ref, uf0_ref, uf1_ref,
               uh0_ref, uh1_ref, out_ref, l10_ref, scre_ref, scro_ref,
               c_ref, *, zbase, nsteps):
    i = pl.program_id(0)
    wbase = tbl_ref[i, 1]
    rbase = tbl_ref[i, 2]
    row_limit = tbl_ref[i, 3]

    @pl.when(i == 0)
    def _():
        scre_ref[pl.ds(zbase, _B), :] = jnp.zeros((_B, _H), jnp.float32)
        scro_ref[pl.ds(zbase, _B), :] = jnp.zeros((_B, _H), jnp.float32)

    xa = xa_ref[...]
    xb = xb_ref[...]
    xl = jnp.concatenate([xa[7:, :], xb[:_B - 1, :]], axis=0)
    wx = jnp.dot(xl, wwt_ref[...],
                 preferred_element_type=jnp.float32) + wb_ref[0:1, :]
    whx = wx[:, :_H]
    wfx = wx[:, _H:]

    h0 = scre_ref[pl.ds(rbase, _B), :]
    h1 = scro_ref[pl.ds(rbase, _B), :]

    fpre = (jnp.dot(h0, uf0_ref[...], preferred_element_type=jnp.float32) +
            jnp.dot(h1, uf1_ref[...], preferred_element_type=jnp.float32))
    f = 0.5 * jnp.tanh(0.5 * (fpre + wfx)) + 0.5
    hcand = jnp.tanh(whx +
                     jnp.dot(f * h0, uh0_ref[...],
                             preferred_element_type=jnp.float32) +
                     jnp.dot(f * h1, uh1_ref[...],
                             preferred_element_type=jnp.float32))
    hn = hcand + f * (h0 + h1 - hcand)
    rows = jax.lax.broadcasted_iota(jnp.int32, (_B, 1), 0)
    hn = jnp.where(rows < row_limit, hn, 0.0)

    hsplit = hn.reshape(_B // 2, 2, _H)
    scre_ref[pl.ds(wbase, _B // 2), :] = hsplit[:, 0, :]
    scro_ref[pl.ds(wbase, _B // 2), :] = hsplit[:, 1, :]

    prev0 = c_ref[0:1, :]
    out_ref[...] = jnp.concatenate([hn[1:, :], prev0], axis=0)
    c_ref[0:1, :] = hn[0:1, :]

    @pl.when(i == nsteps - 1)
    def _():
        l10_ref[...] = hn


def _top_body(x_ref, hbuf_ref, slab_ref, wwt_ref, wb_ref, uf0_ref, uf1_ref,
              uh0_ref, uh1_ref, out_ref, *, levels):
    del hbuf_ref
    xb = x_ref[...]
    wwt = wwt_ref[...]
    wb = wb_ref[0:1, :]
    uf0 = uf0_ref[...]
    uf1 = uf1_ref[...]
    uh0 = uh0_ref[...]
    uh1 = uh1_ref[...]
    hp = slab_ref[...]
    out_ref[pl.ds(_B - 1, 1), :] = hp[0:1, :]
    for l in range(9, -1, -1):
        s, n = levels[l]
        np8 = max(16, n)
        need = 2 * np8
        if hp.shape[0] < need:
            hp = jnp.concatenate(
                [hp, jnp.zeros((need - hp.shape[0], _H), jnp.float32)], axis=0)
        pairs = hp[:need].reshape(np8, 2, _H)
        h0 = pairs[:, 0, :]
        h1 = pairs[:, 1, :]
        xl = xb[s:s + np8, :]
        wx = jnp.dot(xl, wwt, preferred_element_type=jnp.float32) + wb
        whx = wx[:, :_H]
        wfx = wx[:, _H:]
        fpre = (jnp.dot(h0, uf0, preferred_element_type=jnp.float32) +
                jnp.dot(h1, uf1, preferred_element_type=jnp.float32))
        f = 0.5 * jnp.tanh(0.5 * (fpre + wfx)) + 0.5
        hcand = jnp.tanh(whx +
                         jnp.dot(f * h0, uh0,
                                 preferred_element_type=jnp.float32) +
                         jnp.dot(f * h1, uh1,
                                 preferred_element_type=jnp.float32))
        hn = hcand + f * (h0 + h1 - hcand)
        rows = jax.lax.broadcasted_iota(jnp.int32, (np8, 1), 0)
        hn = jnp.where(rows < n, hn, 0.0)
        out_ref[pl.ds(s, n), :] = hn[:n, :]
        hp = hn


def kernel(x, W_w, W_b, U_f, U_h):
    n_nodes = x.shape[0]
    max_level, levels = _plan(n_nodes)
    assert max_level >= 10

    wwt = W_w.T
    wb8 = jnp.tile(W_b[None, :], (8, 1))
    uf0 = U_f[:, :_H].T
    uf1 = U_f[:, _H:].T
    uh0 = U_h[:, :_H].T
    uh1 = U_h[:, _H:].T

    nbs = {lvl: -(-levels[lvl][1] // _B) for lvl in range(max_level, 9, -1)}

    capE = max(nbs[lvl] * _B // 2 for lvl in range(max_level, 9, -1))
    zbase = 2 * capE

    tbl = []
    for lvl in range(max_level, 9, -1):
        s, n = levels[lvl]
        pcur = (max_level - lvl) % 2
        child_half = nbs[lvl + 1] * _B // 2 if lvl < max_level else 0
        for j in range(nbs[lvl] - 1, -1, -1):
            xk = (s + 1) // _B + j
            hasc = lvl < max_level and (j + 1) * _B <= child_half
            row_limit = min(_B, n - j * _B)
            tbl.append([xk, pcur * capE + j * (_B // 2),
                        ((1 - pcur) * capE + j * _B) if hasc else zbase,
                        row_limit])
    tbl = np.asarray(tbl, dtype=np.int32)
    nsteps = tbl.shape[0]

    grid_spec = pltpu.PrefetchScalarGridSpec(
        num_scalar_prefetch=1,
        grid=(nsteps,),
        in_specs=[
            pl.BlockSpec((8, _H), lambda i, t: (128 * t[i, 0] - 1, 0)),
            pl.BlockSpec((_B, _H),
                         lambda i, t, m=(n_nodes - 1) // _B:
                         (jnp.minimum(t[i, 0], m), 0)),
            pl.BlockSpec((_H, 2 * _H), lambda i, t: (0, 0)),
            pl.BlockSpec((8, 2 * _H), lambda i, t: (0, 0)),
            pl.BlockSpec((_H, _H), lambda i, t: (0, 0)),
            pl.BlockSpec((_H, _H), lambda i, t: (0, 0)),
            pl.BlockSpec((_H, _H), lambda i, t: (0, 0)),
            pl.BlockSpec((_H, _H), lambda i, t: (0, 0)),
        ],
        out_specs=[
            pl.BlockSpec((_B, _H), lambda i, t: (t[i, 0], 0)),
            pl.BlockSpec((_B, _H), lambda i, t: (0, 0)),
        ],
        scratch_shapes=[
            pltpu.VMEM((2 * capE + _B, _H), jnp.float32),
            pltpu.VMEM((2 * capE + _B, _H), jnp.float32),
            pltpu.VMEM((8, _H), jnp.float32),
        ],
    )

    h_buf, lvl10 = pl.pallas_call(
        functools.partial(_mega_body, zbase=zbase, nsteps=nsteps),
        grid_spec=grid_spec,
        out_shape=[jax.ShapeDtypeStruct((n_nodes, _H), jnp.float32),
                   jax.ShapeDtypeStruct((_B, _H), jnp.float32)],
    )(tbl, x, x, wwt, wb8, uf0, uf1, uh0, uh1)

    out = pl.pallas_call(
        functools.partial(_top_body, levels=tuple(levels)),
        grid=(1,),
        in_specs=[
            pl.BlockSpec((_B, _H), lambda i: (0, 0)),
            pl.BlockSpec((_B, _H), lambda i: (0, 0)),
            pl.BlockSpec((_B, _H), lambda i: (0, 0)),
            pl.BlockSpec((_H, 2 * _H), lambda i: (0, 0)),
            pl.BlockSpec((8, 2 * _H), lambda i: (0, 0)),
            pl.BlockSpec((_H, _H), lambda i: (0, 0)),
            pl.BlockSpec((_H, _H), lambda i: (0, 0)),
            pl.BlockSpec((_H, _H), lambda i: (0, 0)),
            pl.BlockSpec((_H, _H), lambda i: (0, 0)),
        ],
        out_specs=pl.BlockSpec((_B, _H), lambda i: (0, 0)),
        out_shape=jax.ShapeDtypeStruct((n_nodes, _H), jnp.float32),
        input_output_aliases={1: 0},
    )(x, h_buf, lvl10, wwt, wb8, uf0, uf1, uh0, uh1)
    return out
